# baseline (speedup 1.0000x reference)
import functools

import jax
import jax.numpy as jnp
from jax import lax
from jax.experimental import pallas as pl
from jax.experimental.pallas import tpu as pltpu
from jax.experimental.pallas import tpu_sc as plsc

_F32 = jnp.float32
_BF16 = jnp.bfloat16

_NC = 2
_NS = 16
_NW = _NC * _NS


def _silu(g):
    return g * (1.0 / (1.0 + jnp.exp(-g)))


def _base_body(x_ref, wg_ref, wu_ref, wd_ref, o_ref):
    f = pl.program_id(0)
    g = jnp.dot(x_ref[...], wg_ref[...], preferred_element_type=_F32)
    u = jnp.dot(x_ref[...], wu_ref[...], preferred_element_type=_F32)
    h = (_silu(g) * u).astype(_BF16)
    contrib = jnp.dot(h, wd_ref[...], preferred_element_type=_F32)

    @pl.when(f == 0)
    def _():
        o_ref[...] = contrib

    @pl.when(f > 0)
    def _():
        o_ref[...] += contrib


def _pos_body(ids_ref, pos_ref, counts_ref):
    T = ids_ref.shape[1]
    A = counts_ref.shape[0]
    ids = ids_ref[...]
    a_col = lax.broadcasted_iota(jnp.int32, (A, T), 0)
    onehot = (jnp.broadcast_to(ids, (A, T)) == a_col).astype(_BF16)
    r_io = lax.broadcasted_iota(jnp.int32, (T, T), 0)
    c_io = lax.broadcasted_iota(jnp.int32, (T, T), 1)
    lower_tri = (r_io <= c_io).astype(_BF16)
    csum = jnp.dot(onehot, lower_tri, preferred_element_type=_F32)
    counts = csum[:, T - 1:T]
    a_r = lax.broadcasted_iota(jnp.int32, (A, A), 0)
    a_c = lax.broadcasted_iota(jnp.int32, (A, A), 1)
    strict = (a_c < a_r).astype(_F32)
    offs = jnp.dot(strict, counts, preferred_element_type=_F32)
    z = onehot.astype(_F32) * (csum - 1.0 + offs)
    pos_ref[...] = jnp.sum(z, axis=0, keepdims=True).astype(jnp.int32)
    counts_ref[...] = counts.astype(jnp.int32)


def _adapter_body(offs_ref, elo_ref, ehi_ref, xs_ref, wg_ref, wu_ref,
                  wd_ref, sv_ref, ys_ref):
    t = pl.program_id(0)
    TT, H = xs_ref.shape
    N2 = wg_ref.shape[2]
    xb = xs_ref[...].astype(_BF16)
    row = lax.broadcasted_iota(jnp.int32, (TT, 1), 0) + t * TT

    def body(e, acc):
        g = jnp.dot(xb, wg_ref[e], preferred_element_type=_F32)
        u = jnp.dot(xb, wu_ref[e], preferred_element_type=_F32)
        inter = _silu(g) * u * sv_ref[e]
        m = (row >= offs_ref[e]) & (row < offs_ref[e + 1])
        inter = jnp.where(m, inter, 0.0).astype(_BF16)
        return acc + jnp.dot(inter, wd_ref[e], preferred_element_type=_F32)

    acc0 = jnp.zeros((TT, H), dtype=_F32)
    ys_ref[...] = lax.fori_loop(elo_ref[t], ehi_ref[t] + 1, body, acc0)


def _sc_permute(x_hbm, pos_hbm, xs_hbm, idx_v, rows_v, sem):
    wid = lax.axis_index("s") * _NC + lax.axis_index("c")
    cb = x_hbm.shape[0] // _NW
    base = wid * cb
    pltpu.sync_copy(pos_hbm.at[pl.ds(base, cb)], idx_v)
    pltpu.sync_copy(x_hbm.at[pl.ds(base, cb)], rows_v)
    pltpu.async_copy(rows_v, xs_hbm.at[idx_v], sem).wait()


def _sc_finalize(base_hbm, ys_hbm, pos_hbm, out_hbm, idx_v, yrows_v,
                 brows_v, sem):
    wid = lax.axis_index("s") * _NC + lax.axis_index("c")
    T, H = base_hbm.shape
    cb = T // (_NW * 2)
    nvec = (cb * H) // 16

    for j in range(2):
        sb = wid * cb * 2 + j * cb
        pltpu.sync_copy(pos_hbm.at[pl.ds(sb, cb)], idx_v)
        pltpu.async_copy(ys_hbm.at[idx_v], yrows_v, sem).wait()
        pltpu.sync_copy(base_hbm.at[pl.ds(sb, cb)], brows_v)

        def add_body(i, carry):
            r = i // (H // 16)
            c = (i % (H // 16)) * 16
            brows_v[r, pl.ds(c, 16)] = (brows_v[r, pl.ds(c, 16)]
                                        + yrows_v[r, pl.ds(c, 16)])
            return carry

        lax.fori_loop(0, nvec, add_body, 0)
        pltpu.sync_copy(brows_v, out_hbm.at[pl.ds(sb, cb)])


def kernel(x, w_gate_up, w_down, retain_gate, retain_up, retain_down,
           forget_gate, forget_up, forget_down, scales, token_experiment_ids):
    T, H = x.shape
    FF = w_down.shape[0]
    A, NR, _ = retain_gate.shape
    NFG = forget_gate.shape[1]
    N2 = NR + NFG
    TT = 256
    NT = T // TT

    xb = x.astype(_BF16)
    wg = w_gate_up[:, :FF].astype(_BF16)
    wu = w_gate_up[:, FF:].astype(_BF16)
    wd = w_down.astype(_BF16)
    ids_row = token_experiment_ids.astype(jnp.int32).reshape(1, T)

    pos_row, counts_col = pl.pallas_call(
        _pos_body,
        grid=(1,),
        in_specs=[pl.BlockSpec((1, T), lambda i: (0, 0))],
        out_specs=[pl.BlockSpec((1, T), lambda i: (0, 0)),
                   pl.BlockSpec((A, 1), lambda i: (0, 0))],
        out_shape=[jax.ShapeDtypeStruct((1, T), jnp.int32),
                   jax.ShapeDtypeStruct((A, 1), jnp.int32)],
    )(ids_row)
    pos = pos_row.reshape(T)
    counts = counts_col.reshape(A)
    offs = jnp.concatenate([jnp.zeros((1,), jnp.int32),
                            jnp.cumsum(counts).astype(jnp.int32)])
    tile_starts = jnp.arange(NT, dtype=jnp.int32) * TT
    e_lo = (jnp.searchsorted(offs, tile_starts, side='right') - 1).astype(jnp.int32)
    e_hi = (jnp.searchsorted(offs, tile_starts + TT - 1, side='right') - 1).astype(jnp.int32)
    e_lo = jnp.clip(e_lo, 0, A - 1)
    e_hi = jnp.clip(e_hi, 0, A - 1)

    sc_permute = functools.partial(
        pl.kernel,
        out_type=jax.ShapeDtypeStruct((T, H), _F32),
        mesh=plsc.VectorSubcoreMesh(core_axis_name="c", subcore_axis_name="s"),
        scratch_types=[
            pltpu.VMEM((T // _NW,), jnp.int32),
            pltpu.VMEM((T // _NW, H), _F32),
            pltpu.SemaphoreType.DMA,
        ],
    )(_sc_permute)
    xs = sc_permute(x, pos)

    NFB = 8
    BF = FF // NFB
    base = pl.pallas_call(
        _base_body,
        grid=(NFB,),
        in_specs=[
            pl.BlockSpec((T, H), lambda f: (0, 0)),
            pl.BlockSpec((H, BF), lambda f: (0, f)),
            pl.BlockSpec((H, BF), lambda f: (0, f)),
            pl.BlockSpec((BF, H), lambda f: (f, 0)),
        ],
        out_specs=pl.BlockSpec((T, H), lambda f: (0, 0)),
        out_shape=jax.ShapeDtypeStruct((T, H), _F32),
        compiler_params=pltpu.CompilerParams(
            dimension_semantics=("arbitrary",)),
    )(xb, wg, wu, wd)

    Wg_a = jnp.concatenate([retain_gate, forget_gate], axis=1).transpose(0, 2, 1).astype(_BF16)
    Wu_a = jnp.concatenate([retain_up, forget_up], axis=1).transpose(0, 2, 1).astype(_BF16)
    Wd_a = jnp.concatenate([retain_down.transpose(0, 2, 1),
                            forget_down.transpose(0, 2, 1)], axis=1).astype(_BF16)
    scale_vec = jnp.concatenate([jnp.repeat(scales[:, 0:1], NR, axis=1),
                                 jnp.repeat(scales[:, 1:2], NFG, axis=1)], axis=1)
    scale_vec = scale_vec.reshape(A, 1, N2)

    ys = pl.pallas_call(
        _adapter_body,
        grid=(NT,),
        in_specs=[
            pl.BlockSpec(memory_space=pltpu.SMEM),
            pl.BlockSpec(memory_space=pltpu.SMEM),
            pl.BlockSpec(memory_space=pltpu.SMEM),
            pl.BlockSpec((TT, H), lambda t: (t, 0)),
            pl.BlockSpec((A, H, N2), lambda t: (0, 0, 0)),
            pl.BlockSpec((A, H, N2), lambda t: (0, 0, 0)),
            pl.BlockSpec((A, N2, H), lambda t: (0, 0, 0)),
            pl.BlockSpec((A, 1, N2), lambda t: (0, 0, 0)),
        ],
        out_specs=pl.BlockSpec((TT, H), lambda t: (t, 0)),
        out_shape=jax.ShapeDtypeStruct((T, H), _F32),
        compiler_params=pltpu.CompilerParams(
            dimension_semantics=("arbitrary",)),
    )(offs, e_lo, e_hi, xs, Wg_a, Wu_a, Wd_a, scale_vec)

    sc_finalize = functools.partial(
        pl.kernel,
        out_type=jax.ShapeDtypeStruct((T, H), _F32),
        mesh=plsc.VectorSubcoreMesh(core_axis_name="c", subcore_axis_name="s"),
        scratch_types=[
            pltpu.VMEM((T // (_NW * 2),), jnp.int32),
            pltpu.VMEM((T // (_NW * 2), H), _F32),
            pltpu.VMEM((T // (_NW * 2), H), _F32),
            pltpu.SemaphoreType.DMA,
        ],
    )(_sc_finalize)
    return sc_finalize(base, ys, pos)


# speedup vs baseline: 1.0506x; 1.0506x over previous
import functools

import jax
import jax.numpy as jnp
from jax import lax
from jax.experimental import pallas as pl
from jax.experimental.pallas import tpu as pltpu
from jax.experimental.pallas import tpu_sc as plsc

_F32 = jnp.float32
_BF16 = jnp.bfloat16

_NC = 2
_NS = 16
_NW = _NC * _NS


def _silu(g):
    return g * (1.0 / (1.0 + jnp.exp(-g)))


def _base_body(x_ref, wg_ref, wu_ref, wd_ref, o_ref):
    f = pl.program_id(0)
    g = jnp.dot(x_ref[...], wg_ref[...], preferred_element_type=_F32)
    u = jnp.dot(x_ref[...], wu_ref[...], preferred_element_type=_F32)
    h = (_silu(g) * u).astype(_BF16)
    contrib = jnp.dot(h, wd_ref[...], preferred_element_type=_F32)

    @pl.when(f == 0)
    def _():
        o_ref[...] = contrib

    @pl.when(f > 0)
    def _():
        o_ref[...] += contrib


def _pos_body(ids_ref, pos_ref, ends_ref, *, A):
    NB, L = ids_ref.shape
    R = A * NB
    ids = ids_ref[...]
    ids_t = jnp.concatenate([ids] * A, axis=0)
    a_row = lax.broadcasted_iota(jnp.int32, (R, L), 0) // NB
    onehot = (ids_t == a_row).astype(_BF16)
    c_lo = lax.broadcasted_iota(jnp.int32, (L, L), 0)
    c_hi = lax.broadcasted_iota(jnp.int32, (L, L), 1)
    lt_incl = (c_lo <= c_hi).astype(_BF16)
    csum = jnp.dot(onehot, lt_incl, preferred_element_type=_F32)
    tot = csum[:, L - 1:L]
    r_io = lax.broadcasted_iota(jnp.int32, (R, R), 0)
    c_io = lax.broadcasted_iota(jnp.int32, (R, R), 1)
    same_a = (r_io // NB) == (c_io // NB)
    m_chunk = (same_a & (c_io < r_io)).astype(_F32)
    m_adapt = ((c_io // NB) < (r_io // NB)).astype(_F32)
    cum_c = jnp.dot(m_chunk, tot, preferred_element_type=_F32)
    offs_c = jnp.dot(m_adapt, tot, preferred_element_type=_F32)
    z = onehot.astype(_F32) * (csum + cum_c + offs_c - 1.0)
    acc = z[0:NB]
    for a in range(1, A):
        acc = acc + z[a * NB:(a + 1) * NB]
    pos_ref[...] = acc.astype(jnp.int32)
    ends_ref[...] = (cum_c + tot).astype(jnp.int32)


def _adapter_body(offs_ref, elo_ref, ehi_ref, xs_ref, wg_ref, wu_ref,
                  wd_ref, sv_ref, ys_ref):
    t = pl.program_id(0)
    TT, H = xs_ref.shape
    N2 = wg_ref.shape[2]
    xb = xs_ref[...].astype(_BF16)
    row = lax.broadcasted_iota(jnp.int32, (TT, 1), 0) + t * TT

    def body(e, acc):
        g = jnp.dot(xb, wg_ref[e], preferred_element_type=_F32)
        u = jnp.dot(xb, wu_ref[e], preferred_element_type=_F32)
        inter = _silu(g) * u * sv_ref[e]
        m = (row >= offs_ref[e]) & (row < offs_ref[e + 1])
        inter = jnp.where(m, inter, 0.0).astype(_BF16)
        return acc + jnp.dot(inter, wd_ref[e], preferred_element_type=_F32)

    acc0 = jnp.zeros((TT, H), dtype=_F32)
    ys_ref[...] = lax.fori_loop(elo_ref[t], ehi_ref[t] + 1, body, acc0)


def _sc_permute(x_hbm, pos_hbm, xs_hbm, idx_v, rows_v, sem):
    wid = lax.axis_index("s") * _NC + lax.axis_index("c")
    cb = x_hbm.shape[0] // _NW
    base = wid * cb
    pltpu.sync_copy(pos_hbm.at[pl.ds(base, cb)], idx_v)
    pltpu.sync_copy(x_hbm.at[pl.ds(base, cb)], rows_v)
    pltpu.async_copy(rows_v, xs_hbm.at[idx_v], sem).wait()


def _sc_finalize(base_hbm, ys_hbm, pos_hbm, out_hbm, idx_v, yrows_v,
                 brows_v, sem):
    wid = lax.axis_index("s") * _NC + lax.axis_index("c")
    T, H = base_hbm.shape
    cb = T // (_NW * 2)

    for j in range(2):
        sb = wid * cb * 2 + j * cb
        pltpu.sync_copy(pos_hbm.at[pl.ds(sb, cb)], idx_v)
        pltpu.async_copy(ys_hbm.at[idx_v], yrows_v, sem).wait()
        pltpu.sync_copy(base_hbm.at[pl.ds(sb, cb)], brows_v)

        def add_body(r, carry):
            for c in range(0, H, 16):
                brows_v[r, pl.ds(c, 16)] = (brows_v[r, pl.ds(c, 16)]
                                            + yrows_v[r, pl.ds(c, 16)])
            return carry

        lax.fori_loop(0, cb, add_body, 0)
        pltpu.sync_copy(brows_v, out_hbm.at[pl.ds(sb, cb)])


def kernel(x, w_gate_up, w_down, retain_gate, retain_up, retain_down,
           forget_gate, forget_up, forget_down, scales, token_experiment_ids):
    T, H = x.shape
    FF = w_down.shape[0]
    A, NR, _ = retain_gate.shape
    NFG = forget_gate.shape[1]
    N2 = NR + NFG
    TT = 256
    NT = T // TT

    xb = x.astype(_BF16)
    wg = w_gate_up[:, :FF].astype(_BF16)
    wu = w_gate_up[:, FF:].astype(_BF16)
    wd = w_down.astype(_BF16)
    NB = T // 128
    ids_2d = token_experiment_ids.astype(jnp.int32).reshape(NB, 128)

    pos_2d, ends_col = pl.pallas_call(
        functools.partial(_pos_body, A=A),
        grid=(1,),
        in_specs=[pl.BlockSpec((NB, 128), lambda i: (0, 0))],
        out_specs=[pl.BlockSpec((NB, 128), lambda i: (0, 0)),
                   pl.BlockSpec((A * NB, 1), lambda i: (0, 0))],
        out_shape=[jax.ShapeDtypeStruct((NB, 128), jnp.int32),
                   jax.ShapeDtypeStruct((A * NB, 1), jnp.int32)],
    )(ids_2d)
    pos = pos_2d.reshape(T)
    counts = ends_col.reshape(A, NB)[:, NB - 1]
    offs = jnp.concatenate([jnp.zeros((1,), jnp.int32),
                            jnp.cumsum(counts).astype(jnp.int32)])
    tile_starts = jnp.arange(NT, dtype=jnp.int32) * TT
    e_lo = (jnp.searchsorted(offs, tile_starts, side='right') - 1).astype(jnp.int32)
    e_hi = (jnp.searchsorted(offs, tile_starts + TT - 1, side='right') - 1).astype(jnp.int32)
    e_lo = jnp.clip(e_lo, 0, A - 1)
    e_hi = jnp.clip(e_hi, 0, A - 1)

    sc_permute = functools.partial(
        pl.kernel,
        out_type=jax.ShapeDtypeStruct((T, H), _F32),
        mesh=plsc.VectorSubcoreMesh(core_axis_name="c", subcore_axis_name="s"),
        scratch_types=[
            pltpu.VMEM((T // _NW,), jnp.int32),
            pltpu.VMEM((T // _NW, H), _F32),
            pltpu.SemaphoreType.DMA,
        ],
    )(_sc_permute)
    xs = sc_permute(x, pos)

    NFB = 8
    BF = FF // NFB
    base = pl.pallas_call(
        _base_body,
        grid=(NFB,),
        in_specs=[
            pl.BlockSpec((T, H), lambda f: (0, 0)),
            pl.BlockSpec((H, BF), lambda f: (0, f)),
            pl.BlockSpec((H, BF), lambda f: (0, f)),
            pl.BlockSpec((BF, H), lambda f: (f, 0)),
        ],
        out_specs=pl.BlockSpec((T, H), lambda f: (0, 0)),
        out_shape=jax.ShapeDtypeStruct((T, H), _F32),
        compiler_params=pltpu.CompilerParams(
            dimension_semantics=("arbitrary",)),
    )(xb, wg, wu, wd)

    Wg_a = jnp.concatenate([retain_gate, forget_gate], axis=1).transpose(0, 2, 1).astype(_BF16)
    Wu_a = jnp.concatenate([retain_up, forget_up], axis=1).transpose(0, 2, 1).astype(_BF16)
    Wd_a = jnp.concatenate([retain_down.transpose(0, 2, 1),
                            forget_down.transpose(0, 2, 1)], axis=1).astype(_BF16)
    scale_vec = jnp.concatenate([jnp.repeat(scales[:, 0:1], NR, axis=1),
                                 jnp.repeat(scales[:, 1:2], NFG, axis=1)], axis=1)
    scale_vec = scale_vec.reshape(A, 1, N2)

    ys = pl.pallas_call(
        _adapter_body,
        grid=(NT,),
        in_specs=[
            pl.BlockSpec(memory_space=pltpu.SMEM),
            pl.BlockSpec(memory_space=pltpu.SMEM),
            pl.BlockSpec(memory_space=pltpu.SMEM),
            pl.BlockSpec((TT, H), lambda t: (t, 0)),
            pl.BlockSpec((A, H, N2), lambda t: (0, 0, 0)),
            pl.BlockSpec((A, H, N2), lambda t: (0, 0, 0)),
            pl.BlockSpec((A, N2, H), lambda t: (0, 0, 0)),
            pl.BlockSpec((A, 1, N2), lambda t: (0, 0, 0)),
        ],
        out_specs=pl.BlockSpec((TT, H), lambda t: (t, 0)),
        out_shape=jax.ShapeDtypeStruct((T, H), _F32),
        compiler_params=pltpu.CompilerParams(
            dimension_semantics=("arbitrary",)),
    )(offs, e_lo, e_hi, xs, Wg_a, Wu_a, Wd_a, scale_vec)

    sc_finalize = functools.partial(
        pl.kernel,
        out_type=jax.ShapeDtypeStruct((T, H), _F32),
        mesh=plsc.VectorSubcoreMesh(core_axis_name="c", subcore_axis_name="s"),
        scratch_types=[
            pltpu.VMEM((T // (_NW * 2),), jnp.int32),
            pltpu.VMEM((T // (_NW * 2), H), _F32),
            pltpu.VMEM((T // (_NW * 2), H), _F32),
            pltpu.SemaphoreType.DMA,
        ],
    )(_sc_finalize)
    return sc_finalize(base, ys, pos)
